# initial kernel scaffold (unmeasured)
import jax
import jax.numpy as jnp
from jax import lax
from jax.experimental import pallas as pl
from jax.experimental.pallas import tpu as pltpu

N_DEV = 8


def kernel(x, w_mat):
    M, K_shard = x.shape
    _, N = w_mat.shape
    CH = M // N_DEV

    def body(x_ref, w_ref, out_ref,
             xbf_ref, wbf_ref, comm_ref, ax_src_ref, ax_dst_ref,
             send_sems, recv_sems, ax_send_sems, ax_recv_sems):
        my = lax.axis_index("i")
        left = lax.rem(my + N_DEV - 1, N_DEV)
        right = lax.rem(my + 1, N_DEV)

        barrier_sem = pltpu.get_barrier_semaphore()
        for nbr in (left, right):
            pl.semaphore_signal(barrier_sem, inc=1, device_id=(nbr,),
                                device_id_type=pl.DeviceIdType.MESH)
        pl.semaphore_wait(barrier_sem, 2)

        xbf_ref[...] = x_ref[...].astype(jnp.bfloat16)
        wbf_ref[...] = w_ref[...].astype(jnp.bfloat16)

        def local_chunk(j):
            rows = xbf_ref[pl.ds(j * CH, CH), :]
            return jnp.dot(rows, wbf_ref[...],
                           preferred_element_type=jnp.float32)

        def ring_send(t, src_slot):
            return pltpu.make_async_remote_copy(
                src_ref=comm_ref.at[src_slot],
                dst_ref=comm_ref.at[t],
                send_sem=send_sems.at[t],
                recv_sem=recv_sems.at[t],
                device_id=(right,),
                device_id_type=pl.DeviceIdType.MESH,
            )

        j0 = lax.rem(my + N_DEV - 1, N_DEV)
        comm_ref[N_DEV - 1, :, :] = local_chunk(j0)
        rdma = ring_send(0, N_DEV - 1)
        rdma.start()
        rdma.wait()

        for t in range(1, N_DEV - 1):
            j = lax.rem(my + 2 * N_DEV - t - 1, N_DEV)
            comm_ref[t - 1, :, :] = comm_ref[t - 1, :, :] + local_chunk(j)
            rdma = ring_send(t, t - 1)
            rdma.start()
            rdma.wait()

        y = comm_ref[N_DEV - 2, :, :] + local_chunk(my)
        y = jnp.maximum(y, 0.0)
        out_ref[...] = y

        a = jnp.max(y)
        ax_src_ref[...] = jnp.full((1, 128), a, jnp.float32)
        ax_dst_ref[pl.ds(my, 1), :] = ax_src_ref[...]

        ax_sends = []
        for k in range(1, N_DEV):
            tgt = lax.rem(my + k, N_DEV)
            s = pltpu.make_async_remote_copy(
                src_ref=ax_src_ref,
                dst_ref=ax_dst_ref.at[pl.ds(my, 1), :],
                send_sem=ax_send_sems.at[k],
                recv_sem=ax_recv_sems.at[my],
                device_id=(tgt,),
                device_id_type=pl.DeviceIdType.MESH,
            )
            s.start()
            ax_sends.append(s)

        for d in range(N_DEV):
            @pl.when(my != d)
            def _():
                r = pltpu.make_async_remote_copy(
                    src_ref=ax_src_ref,
                    dst_ref=ax_dst_ref.at[pl.ds(d, 1), :],
                    send_sem=ax_send_sems.at[0],
                    recv_sem=ax_recv_sems.at[d],
                    device_id=(left,),
                    device_id_type=pl.DeviceIdType.MESH,
                )
                r.wait_recv()

        g = jnp.max(ax_dst_ref[...])

        scale = g / 448.0
        inv = 448.0 / g
        q = jnp.clip(out_ref[...] * inv, 0.0, 448.0)
        q = q.astype(jnp.float8_e4m3fn).astype(jnp.float32)
        out_ref[...] = q * scale

        for s in ax_sends:
            s.wait_send()

    return pl.pallas_call(
        body,
        out_shape=jax.ShapeDtypeStruct((CH, N), jnp.float32),
        in_specs=[pl.BlockSpec(memory_space=pltpu.VMEM),
                  pl.BlockSpec(memory_space=pltpu.VMEM)],
        out_specs=pl.BlockSpec(memory_space=pltpu.VMEM),
        scratch_shapes=[
            pltpu.VMEM((M, K_shard), jnp.bfloat16),
            pltpu.VMEM((K_shard, N), jnp.bfloat16),
            pltpu.VMEM((N_DEV, CH, N), jnp.float32),
            pltpu.VMEM((1, 128), jnp.float32),
            pltpu.VMEM((N_DEV, 128), jnp.float32),
            pltpu.SemaphoreType.DMA((N_DEV - 1,)),
            pltpu.SemaphoreType.DMA((N_DEV - 1,)),
            pltpu.SemaphoreType.DMA((N_DEV,)),
            pltpu.SemaphoreType.DMA((N_DEV,)),
        ],
        compiler_params=pltpu.CompilerParams(collective_id=0),
    )(x, w_mat)


# baseline (device time: 358322 ns/iter reference)
import jax
import jax.numpy as jnp
from jax import lax
from jax.experimental import pallas as pl
from jax.experimental.pallas import tpu as pltpu

N_DEV = 8


def kernel(x, w_mat):
    M, K_shard = x.shape
    _, N = w_mat.shape
    CH = M // N_DEV

    def body(x_ref, w_ref, out_ref,
             xbf_ref, wbf_ref, comm_ref, ax_src_ref, ax_dst_ref,
             send_sems, recv_sems, ax_send_sems, ax_recv_sems):
        my = lax.axis_index("i")
        left = lax.rem(my + N_DEV - 1, N_DEV)
        right = lax.rem(my + 1, N_DEV)

        barrier_sem = pltpu.get_barrier_semaphore()
        for nbr in (left, right):
            pl.semaphore_signal(barrier_sem, inc=1, device_id=(nbr,),
                                device_id_type=pl.DeviceIdType.MESH)
        pl.semaphore_wait(barrier_sem, 2)

        xbf_ref[...] = x_ref[...].astype(jnp.bfloat16)
        wbf_ref[...] = w_ref[...].astype(jnp.bfloat16)

        def local_chunk(j):
            rows = xbf_ref[pl.ds(j * CH, CH), :]
            return jnp.dot(rows, wbf_ref[...],
                           preferred_element_type=jnp.float32)

        def ring_send(t, src_slot):
            return pltpu.make_async_remote_copy(
                src_ref=comm_ref.at[src_slot],
                dst_ref=comm_ref.at[t],
                send_sem=send_sems.at[t],
                recv_sem=recv_sems.at[t],
                device_id=(right,),
                device_id_type=pl.DeviceIdType.MESH,
            )

        j0 = lax.rem(my + N_DEV - 1, N_DEV)
        comm_ref[N_DEV - 1, :, :] = local_chunk(j0)
        rdma = ring_send(0, N_DEV - 1)
        rdma.start()
        rdma.wait()

        for t in range(1, N_DEV - 1):
            j = lax.rem(my + 2 * N_DEV - t - 1, N_DEV)
            comm_ref[t - 1, :, :] = comm_ref[t - 1, :, :] + local_chunk(j)
            rdma = ring_send(t, t - 1)
            rdma.start()
            rdma.wait()

        y = comm_ref[N_DEV - 2, :, :] + local_chunk(my)
        y = jnp.maximum(y, 0.0)
        out_ref[...] = y

        a = jnp.max(y)
        ax_src_ref[...] = jnp.full((1, 128), a, jnp.float32)
        ax_dst_ref[pl.ds(my, 1), :] = ax_src_ref[...]

        ax_sends = []
        for k in range(1, N_DEV):
            tgt = lax.rem(my + k, N_DEV)
            s = pltpu.make_async_remote_copy(
                src_ref=ax_src_ref,
                dst_ref=ax_dst_ref.at[pl.ds(my, 1), :],
                send_sem=ax_send_sems.at[k],
                recv_sem=ax_recv_sems.at[my],
                device_id=(tgt,),
                device_id_type=pl.DeviceIdType.MESH,
            )
            s.start()
            ax_sends.append(s)

        for d in range(N_DEV):
            @pl.when(my != d)
            def _():
                r = pltpu.make_async_remote_copy(
                    src_ref=ax_src_ref,
                    dst_ref=ax_dst_ref.at[pl.ds(d, 1), :],
                    send_sem=ax_send_sems.at[0],
                    recv_sem=ax_recv_sems.at[d],
                    device_id=(left,),
                    device_id_type=pl.DeviceIdType.MESH,
                )
                r.wait_recv()

        g = jnp.max(ax_dst_ref[...])

        scale = g / 448.0
        inv = 448.0 / g
        q = jnp.clip(out_ref[...] * inv, 0.0, 448.0)
        q = q.astype(jnp.float8_e4m3fn).astype(jnp.float32)
        out_ref[...] = q * scale

        for s in ax_sends:
            s.wait_send()

    return pl.pallas_call(
        body,
        out_shape=jax.ShapeDtypeStruct((CH, N), jnp.float32),
        in_specs=[pl.BlockSpec(memory_space=pltpu.VMEM),
                  pl.BlockSpec(memory_space=pltpu.VMEM)],
        out_specs=pl.BlockSpec(memory_space=pltpu.VMEM),
        scratch_shapes=[
            pltpu.VMEM((M, K_shard), jnp.bfloat16),
            pltpu.VMEM((K_shard, N), jnp.bfloat16),
            pltpu.VMEM((N_DEV, CH, N), jnp.float32),
            pltpu.VMEM((1, 128), jnp.float32),
            pltpu.VMEM((N_DEV, 128), jnp.float32),
            pltpu.SemaphoreType.DMA((N_DEV - 1,)),
            pltpu.SemaphoreType.DMA((N_DEV - 1,)),
            pltpu.SemaphoreType.DMA((N_DEV,)),
            pltpu.SemaphoreType.DMA((N_DEV,)),
        ],
        compiler_params=pltpu.CompilerParams(
            collective_id=0, vmem_limit_bytes=100 * 1024 * 1024),
    )(x, w_mat)


# device time: 156075 ns/iter; 2.2958x vs baseline; 2.2958x over previous
import jax
import jax.numpy as jnp
from jax import lax
from jax.experimental import pallas as pl
from jax.experimental.pallas import tpu as pltpu

N_DEV = 8


def kernel(x, w_mat):
    M, K_shard = x.shape
    _, N = w_mat.shape
    CH = M // N_DEV

    def body(x_ref, w_ref, out_ref,
             xbf_ref, wbf_ref, sendbuf_ref, parts_ref,
             ax_src_ref, ax_dst_ref,
             send_sems, recv_sems, ax_send_sems, ax_recv_sems):
        my = lax.axis_index("i")

        barrier_sem = pltpu.get_barrier_semaphore()
        for k in range(1, N_DEV):
            tgt = lax.rem(my + k, N_DEV)
            pl.semaphore_signal(barrier_sem, inc=1, device_id=(tgt,),
                                device_id_type=pl.DeviceIdType.MESH)
        pl.semaphore_wait(barrier_sem, N_DEV - 1)

        xbf_ref[...] = x_ref[...].astype(jnp.bfloat16)
        wbf_ref[...] = w_ref[...].astype(jnp.bfloat16)

        def partial_chunk(j):
            return jnp.dot(xbf_ref[pl.ds(j * CH, CH), :], wbf_ref[...],
                           preferred_element_type=jnp.float32
                           ).astype(jnp.bfloat16)

        sends = []
        for k in range(1, N_DEV):
            tgt = lax.rem(my + k, N_DEV)
            sendbuf_ref[k, :, :] = partial_chunk(tgt)
            s = pltpu.make_async_remote_copy(
                src_ref=sendbuf_ref.at[k],
                dst_ref=parts_ref.at[my],
                send_sem=send_sems.at[k],
                recv_sem=recv_sems.at[my],
                device_id=(tgt,),
                device_id_type=pl.DeviceIdType.MESH,
            )
            s.start()
            sends.append(s)

        parts_ref[pl.ds(my, 1), :, :] = partial_chunk(my)[None]

        y = jnp.zeros((CH, N), jnp.float32)
        for s in range(N_DEV):
            @pl.when(my != s)
            def _():
                r = pltpu.make_async_remote_copy(
                    src_ref=sendbuf_ref.at[0],
                    dst_ref=parts_ref.at[s],
                    send_sem=send_sems.at[0],
                    recv_sem=recv_sems.at[s],
                    device_id=(my,),
                    device_id_type=pl.DeviceIdType.MESH,
                )
                r.wait_recv()
            y = y + parts_ref[s, :, :].astype(jnp.float32)

        y = jnp.maximum(y, 0.0)
        out_ref[...] = y

        a = jnp.max(y)
        ax_src_ref[...] = jnp.full((1, 128), a, jnp.float32)
        ax_dst_ref[pl.ds(my, 1), :] = ax_src_ref[...]

        ax_sends = []
        for k in range(1, N_DEV):
            tgt = lax.rem(my + k, N_DEV)
            s = pltpu.make_async_remote_copy(
                src_ref=ax_src_ref,
                dst_ref=ax_dst_ref.at[pl.ds(my, 1), :],
                send_sem=ax_send_sems.at[k],
                recv_sem=ax_recv_sems.at[my],
                device_id=(tgt,),
                device_id_type=pl.DeviceIdType.MESH,
            )
            s.start()
            ax_sends.append(s)

        for d in range(N_DEV):
            @pl.when(my != d)
            def _():
                r = pltpu.make_async_remote_copy(
                    src_ref=ax_src_ref,
                    dst_ref=ax_dst_ref.at[pl.ds(d, 1), :],
                    send_sem=ax_send_sems.at[0],
                    recv_sem=ax_recv_sems.at[d],
                    device_id=(my,),
                    device_id_type=pl.DeviceIdType.MESH,
                )
                r.wait_recv()

        g = jnp.max(ax_dst_ref[...])

        scale = g / 448.0
        inv = 448.0 / g
        q = jnp.clip(out_ref[...] * inv, 0.0, 448.0)
        q = q.astype(jnp.float8_e4m3fn).astype(jnp.float32)
        out_ref[...] = q * scale

        for s in sends:
            s.wait_send()
        for s in ax_sends:
            s.wait_send()

    return pl.pallas_call(
        body,
        out_shape=jax.ShapeDtypeStruct((CH, N), jnp.float32),
        in_specs=[pl.BlockSpec(memory_space=pltpu.VMEM),
                  pl.BlockSpec(memory_space=pltpu.VMEM)],
        out_specs=pl.BlockSpec(memory_space=pltpu.VMEM),
        scratch_shapes=[
            pltpu.VMEM((M, K_shard), jnp.bfloat16),
            pltpu.VMEM((K_shard, N), jnp.bfloat16),
            pltpu.VMEM((N_DEV, CH, N), jnp.bfloat16),
            pltpu.VMEM((N_DEV, CH, N), jnp.bfloat16),
            pltpu.VMEM((1, 128), jnp.float32),
            pltpu.VMEM((N_DEV, 128), jnp.float32),
            pltpu.SemaphoreType.DMA((N_DEV,)),
            pltpu.SemaphoreType.DMA((N_DEV,)),
            pltpu.SemaphoreType.DMA((N_DEV,)),
            pltpu.SemaphoreType.DMA((N_DEV,)),
        ],
        compiler_params=pltpu.CompilerParams(
            collective_id=0, vmem_limit_bytes=100 * 1024 * 1024),
    )(x, w_mat)


# device time: 80569 ns/iter; 4.4474x vs baseline; 1.9372x over previous
import jax
import jax.numpy as jnp
from jax import lax
from jax.experimental import pallas as pl
from jax.experimental.pallas import tpu as pltpu

N_DEV = 8

ORDS = (
    (0, 640, ("x", "y", "z")),
    (640, 640, ("y", "z", "x")),
    (1280, 768, ("z", "x", "y")),
)


def kernel(x, w_mat):
    M, K_shard = x.shape
    _, N = w_mat.shape
    CH = M // N_DEV

    def body(x_ref, w_ref, out_ref, *scr):
        (xbf_ref, wbf_ref, pbuf_ref,
         rb1A, rb1B, rb1C, rb2A, rb2B, rb2C, rb3A, rb3B, rb3C,
         sb2A, sb2B, sb2C, sb3A, sb3B, sb3C,
         ax_src_ref, ax_dst_ref,
         s1_sems, r1_sems, s2_sems, r2_sems, s3_sems, r3_sems,
         ax_send_sems, ax_recv_sems) = scr
        rb1 = (rb1A, rb1B, rb1C)
        rb2 = (rb2A, rb2B, rb2C)
        rb3 = (rb3A, rb3B, rb3C)
        sb2 = (sb2A, sb2B, sb2C)
        sb3 = (sb3A, sb3B, sb3C)

        f32 = jnp.float32
        bf16 = jnp.bfloat16
        my = lax.axis_index("i")
        s = lax.rem(my, 4)
        zb = my // 4

        def offs_dev(dx, dy, dz):
            s_ = s
            if dx:
                s_ = jnp.bitwise_xor(s_, 1)
            if dy:
                s_ = 3 - s_
            z_ = (1 - zb) if dz else zb
            return z_ * 4 + s_

        nbr = {"x": offs_dev(1, 0, 0),
               "y": offs_dev(0, 1, 0),
               "z": offs_dev(0, 0, 1)}

        def chunk_dev(dims, e1, e2, e3):
            d = dict(zip(dims, (e1, e2, e3)))
            return offs_dev(d["x"], d["y"], d["z"])

        barrier_sem = pltpu.get_barrier_semaphore()
        for n in nbr.values():
            pl.semaphore_signal(barrier_sem, inc=1, device_id=(n,),
                                device_id_type=pl.DeviceIdType.MESH)
        pl.semaphore_wait(barrier_sem, 3)

        xbf_ref[...] = x_ref[...].astype(bf16)
        wbf_ref[...] = w_ref[...].astype(bf16)

        def pc(dev):
            return jnp.dot(xbf_ref[pl.ds(dev * CH, CH), :], wbf_ref[...],
                           preferred_element_type=f32)

        sends = []

        def send(src, dst, ssem, rsem, tgt):
            d = pltpu.make_async_remote_copy(
                src_ref=src, dst_ref=dst, send_sem=ssem, recv_sem=rsem,
                device_id=(tgt,), device_id_type=pl.DeviceIdType.MESH)
            d.start()
            sends.append(d)

        def wait_recv(dst, rsem):
            pltpu.make_async_remote_copy(
                src_ref=dst, dst_ref=dst, send_sem=rsem, recv_sem=rsem,
                device_id=(my,), device_id_type=pl.DeviceIdType.MESH,
            ).wait_recv()

        rel_order = [(1, 1, 1), (1, 1, 0), (1, 0, 1), (0, 1, 1),
                     (1, 0, 0), (0, 1, 0), (0, 0, 1)]
        for (dx, dy, dz) in rel_order:
            rel = dx * 4 + dy * 2 + dz
            pbuf_ref[rel, :, :] = pc(offs_dev(dx, dy, dz)).astype(bf16)
            d = {"x": dx, "y": dy, "z": dz}
            for o, (c0, w, dims) in enumerate(ORDS):
                e1, e2, e3 = d[dims[0]], d[dims[1]], d[dims[2]]
                if e1 == 1:
                    slot = e2 * 2 + e3
                    send(pbuf_ref.at[rel, :, c0:c0 + w],
                         rb1[o].at[slot],
                         s1_sems.at[o, slot], r1_sems.at[o, slot],
                         nbr[dims[0]])

        for o, (c0, w, dims) in enumerate(ORDS):
            for b in range(2):
                rel = sum(v << sh for v, sh in
                          zip((dict(zip(dims, (0, 1, b)))[k]
                               for k in ("x", "y", "z")), (2, 1, 0)))
                wait_recv(rb1[o].at[2 + b], r1_sems.at[o, 2 + b])
                sb2[o][b, :, :] = (
                    pbuf_ref[rel, :, c0:c0 + w].astype(f32)
                    + rb1[o][2 + b, :, :].astype(f32)).astype(bf16)
                send(sb2[o].at[b], rb2[o].at[b],
                     s2_sems.at[o, b], r2_sems.at[o, b], nbr[dims[1]])

        for o, (c0, w, dims) in enumerate(ORDS):
            rel = sum(v << sh for v, sh in
                      zip((dict(zip(dims, (0, 0, 1)))[k]
                           for k in ("x", "y", "z")), (2, 1, 0)))
            wait_recv(rb1[o].at[1], r1_sems.at[o, 1])
            wait_recv(rb2[o].at[1], r2_sems.at[o, 1])
            sb3[o][...] = (
                pbuf_ref[rel, :, c0:c0 + w].astype(f32)
                + rb1[o][1, :, :].astype(f32)
                + rb2[o][1, :, :].astype(f32)).astype(bf16)
            send(sb3[o], rb3[o], s3_sems.at[o], r3_sems.at[o],
                 nbr[dims[2]])

        cO = pc(my)
        a = jnp.zeros((), f32)
        for o, (c0, w, dims) in enumerate(ORDS):
            wait_recv(rb1[o].at[0], r1_sems.at[o, 0])
            wait_recv(rb2[o].at[0], r2_sems.at[o, 0])
            wait_recv(rb3[o], r3_sems.at[o])
            y = (cO[:, c0:c0 + w]
                 + rb1[o][0, :, :].astype(f32)
                 + rb2[o][0, :, :].astype(f32)
                 + rb3[o][...].astype(f32))
            y = jnp.maximum(y, 0.0)
            out_ref[:, c0:c0 + w] = y
            a = jnp.maximum(a, jnp.max(y))

        ax_src_ref[...] = jnp.full((1, 128), a, f32)
        ax_dst_ref[pl.ds(my, 1), :] = ax_src_ref[...]

        for k in range(1, N_DEV):
            tgt = lax.rem(my + k, N_DEV)
            send(ax_src_ref, ax_dst_ref.at[pl.ds(my, 1), :],
                 ax_send_sems.at[k], ax_recv_sems.at[my], tgt)

        for d in range(N_DEV):
            @pl.when(my != d)
            def _():
                wait_recv(ax_dst_ref.at[pl.ds(d, 1), :],
                          ax_recv_sems.at[d])

        g = jnp.max(ax_dst_ref[...])

        scale = g / 448.0
        inv = 448.0 / g
        q = jnp.clip(out_ref[...] * inv, 0.0, 448.0)
        q = q.astype(jnp.float8_e4m3fn).astype(f32)
        out_ref[...] = q * scale

        for d in sends:
            d.wait_send()

    scratch = [
        pltpu.VMEM((M, K_shard), jnp.bfloat16),
        pltpu.VMEM((K_shard, N), jnp.bfloat16),
        pltpu.VMEM((N_DEV, CH, N), jnp.bfloat16),
    ]
    for _, w, _ in ORDS:
        scratch.append(pltpu.VMEM((4, CH, w), jnp.bfloat16))
    for _, w, _ in ORDS:
        scratch.append(pltpu.VMEM((2, CH, w), jnp.bfloat16))
    for _, w, _ in ORDS:
        scratch.append(pltpu.VMEM((CH, w), jnp.bfloat16))
    for _, w, _ in ORDS:
        scratch.append(pltpu.VMEM((2, CH, w), jnp.bfloat16))
    for _, w, _ in ORDS:
        scratch.append(pltpu.VMEM((CH, w), jnp.bfloat16))
    scratch += [
        pltpu.VMEM((1, 128), jnp.float32),
        pltpu.VMEM((N_DEV, 128), jnp.float32),
        pltpu.SemaphoreType.DMA((3, 4)),
        pltpu.SemaphoreType.DMA((3, 4)),
        pltpu.SemaphoreType.DMA((3, 2)),
        pltpu.SemaphoreType.DMA((3, 2)),
        pltpu.SemaphoreType.DMA((3,)),
        pltpu.SemaphoreType.DMA((3,)),
        pltpu.SemaphoreType.DMA((N_DEV,)),
        pltpu.SemaphoreType.DMA((N_DEV,)),
    ]

    return pl.pallas_call(
        body,
        out_shape=jax.ShapeDtypeStruct((CH, N), jnp.float32),
        in_specs=[pl.BlockSpec(memory_space=pltpu.VMEM),
                  pl.BlockSpec(memory_space=pltpu.VMEM)],
        out_specs=pl.BlockSpec(memory_space=pltpu.VMEM),
        scratch_shapes=scratch,
        compiler_params=pltpu.CompilerParams(
            collective_id=0, vmem_limit_bytes=100 * 1024 * 1024),
    )(x, w_mat)
